# baseline (speedup 1.0000x reference)
import functools
import jax
import jax.numpy as jnp
from jax import lax
from jax.experimental import pallas as pl
from jax.experimental.pallas import tpu as pltpu
from jax.experimental.pallas import tpu_sc as plsc

N = 10000
NP = 10240
D = 128
E = 320000
NC = 2
NS = 16
NW = NC * NS
EPW = E // NW
CH = 80
NCH = EPW // CH
NG = EPW // 16
BLK = 2048
GRID = NP // BLK


def _dense_body(x_ref, wc_ref, wl_ref, asv_ref, adv_ref, bl_ref,
                h_ref, hl_ref, vecs_ref, cvec_ref, ma_ref, mb_ref):
    i = pl.program_id(0)
    xb = x_ref[...]
    h = jnp.dot(xb, wc_ref[...], preferred_element_type=jnp.float32)
    h_ref[...] = h
    hl_ref[...] = (jnp.dot(xb, wl_ref[...], preferred_element_type=jnp.float32)
                   + bl_ref[...][None, :])
    asrc = jnp.dot(h, asv_ref[...], preferred_element_type=jnp.float32)
    adst = jnp.dot(h, adv_ref[...], preferred_element_type=jnp.float32)
    vecs_ref[0, :] = asrc
    vecs_ref[1, :] = adst

    @pl.when(i == 0)
    def _():
        ma_ref[0] = -1e30
        mb_ref[0] = -1e30

    ma_ref[0] = jnp.maximum(ma_ref[0], jnp.max(asrc))
    mb_ref[0] = jnp.maximum(mb_ref[0], jnp.max(adst))

    @pl.when(i == GRID - 1)
    def _():
        c = ma_ref[0] + mb_ref[0]
        c = jnp.where(c > 0, c, 0.2 * c)
        cvec_ref[...] = jnp.full((8, 128), c, jnp.float32)


def _dense(h_in, Wc, Wl, a_src, a_dst, bl):
    return pl.pallas_call(
        _dense_body,
        grid=(GRID,),
        in_specs=[
            pl.BlockSpec((BLK, D), lambda i: (i, 0)),
            pl.BlockSpec((D, D), lambda i: (0, 0)),
            pl.BlockSpec((D, D), lambda i: (0, 0)),
            pl.BlockSpec((D,), lambda i: (0,)),
            pl.BlockSpec((D,), lambda i: (0,)),
            pl.BlockSpec((D,), lambda i: (0,)),
        ],
        out_specs=[
            pl.BlockSpec((BLK, D), lambda i: (i, 0)),
            pl.BlockSpec((BLK, D), lambda i: (i, 0)),
            pl.BlockSpec((8, BLK), lambda i: (0, i)),
            pl.BlockSpec((8, 128), lambda i: (0, 0)),
        ],
        out_shape=[
            jax.ShapeDtypeStruct((NP, D), jnp.float32),
            jax.ShapeDtypeStruct((NP, D), jnp.float32),
            jax.ShapeDtypeStruct((8, NP), jnp.float32),
            jax.ShapeDtypeStruct((8, 128), jnp.float32),
        ],
        scratch_shapes=[pltpu.SMEM((1,), jnp.float32),
                        pltpu.SMEM((1,), jnp.float32)],
    )(h_in, Wc, Wl, a_src, a_dst, bl)


def _dyngather(x, idx):
    dnums = lax.GatherDimensionNumbers(
        offset_dims=(), collapsed_slice_dims=(0,), start_index_map=(0,))
    return lax.gather(x, idx[:, None], dnums, slice_sizes=(1,),
                      mode=lax.GatherScatterMode.PROMISE_IN_BOUNDS)


def _escore_body(vecs_hbm, cvec_hbm, src_hbm, dst_hbm,
                 exw_hbm, dparts_hbm,
                 tbl_a, tbl_b, sidx, didx2, exl, denom, iidx, gbuf, zbuf,
                 dsum_s, dsem):
    cid = lax.axis_index("c")
    sid = lax.axis_index("s")
    wid = sid * NC + cid
    eoff = wid * EPW

    pltpu.make_async_copy(vecs_hbm.at[0], tbl_a, dsem).start()
    pltpu.make_async_copy(vecs_hbm.at[1], tbl_b, dsem).start()
    pltpu.make_async_copy(cvec_hbm.at[0, pl.ds(0, 16)], gbuf, dsem).start()
    pltpu.make_async_copy(src_hbm.at[pl.ds(eoff, EPW)], sidx, dsem).start()

    def _ld_didx(ch, _):
        pltpu.make_async_copy(dst_hbm.at[pl.ds(eoff + ch * CH, CH)],
                              didx2.at[ch], dsem).start()
        return 0
    lax.fori_loop(0, NCH, _ld_didx, 0)

    zero16 = jnp.zeros((16,), jnp.float32)
    io16 = lax.iota(jnp.int32, 16)

    def _zr(k, _):
        for cc in range(8):
            denom[k, pl.ds(cc * 16, 16)] = zero16
        return 0
    lax.fori_loop(0, CH, _zr, 0)
    for k in range(8):
        for cc in range(8):
            zbuf[k, pl.ds(cc * 16, 16)] = zero16
    pltpu.sync_copy(zbuf, dsum_s.at[pl.ds(sid * 8, 8)])
    for j in range(5):
        iidx[0, pl.ds(j * 16, 16)] = io16 + j * 16

    pltpu.make_async_copy(vecs_hbm.at[0], tbl_a, dsem).wait()
    pltpu.make_async_copy(vecs_hbm.at[1], tbl_b, dsem).wait()
    pltpu.make_async_copy(cvec_hbm.at[0, pl.ds(0, 16)], gbuf, dsem).wait()
    pltpu.make_async_copy(src_hbm.at[pl.ds(0, EPW)], sidx, dsem).wait()

    def _dr_didx(ch, _):
        pltpu.make_async_copy(dst_hbm.at[pl.ds(0, CH)], didx2.at[0],
                              dsem).wait()
        return 0
    lax.fori_loop(0, NCH, _dr_didx, 0)
    cconst = gbuf[...]

    plsc.subcore_barrier()

    @plsc.parallel_loop(0, NG, step=1, unroll=8)
    def _grp(i):
        s16 = sidx[pl.ds(i * 16, 16)]
        row = i // 5
        col = (i % 5) * 16
        d16 = didx2[row, pl.ds(col, 16)]
        e = plsc.load_gather(tbl_a, [s16]) + plsc.load_gather(tbl_b, [d16])
        e = jnp.where(e > 0, e, 0.2 * e)
        ex = jnp.exp(e - cconst)
        exl[pl.ds(i * 16, 16)] = ex
        plsc.addupdate_scatter(denom, [d16 >> 7, d16 & 127], ex)

    pltpu.sync_copy(exl, exw_hbm.at[pl.ds(eoff, EPW)])
    pltpu.sync_copy(denom, dsum_s.at[iidx.at[0]], add=True)
    plsc.subcore_barrier()
    pltpu.sync_copy(dsum_s.at[pl.ds(sid * 8, 8)],
                    dparts_hbm.at[cid].at[pl.ds(sid * 8, 8)])


def _escore(vecs, cvec, src, dst):
    mesh = plsc.VectorSubcoreMesh(core_axis_name="c", subcore_axis_name="s")
    fn = pl.kernel(
        _escore_body,
        out_type=[
            jax.ShapeDtypeStruct((E,), jnp.float32),
            jax.ShapeDtypeStruct((NC, 128, D), jnp.float32),
        ],
        mesh=mesh,
        scratch_types=[
            pltpu.VMEM((NP,), jnp.float32),
            pltpu.VMEM((NP,), jnp.float32),
            pltpu.VMEM((EPW,), jnp.int32),
            pltpu.VMEM((NCH, CH), jnp.int32),
            pltpu.VMEM((EPW,), jnp.float32),
            pltpu.VMEM((CH, D), jnp.float32),
            pltpu.VMEM((1, CH), jnp.int32),
            pltpu.VMEM((16,), jnp.float32),
            pltpu.VMEM((8, D), jnp.float32),
            pltpu.VMEM_SHARED((128, D), jnp.float32),
            pltpu.SemaphoreType.DMA,
        ],
        compiler_params=pltpu.CompilerParams(needs_layout_passes=False),
    )
    return fn(vecs, cvec, src, dst)


def _rows_body(h_hbm, exw_hbm, src_hbm, dst_hbm,
               parts_hbm,
               sidxb0, sidxb1, sidxb2, sidxb3,
               didxb0, didxb1, didxb2, didxb3,
               exbb0, exbb1, exbb2, exbb3,
               rowsb0, rowsb1, rowsb2, rowsb3,
               acc_s,
               si0, si1, si2, si3, dd0, dd1, dd2, dd3,
               gg0, gg1, gg2, gg3, ss0, ss1, ss2, ss3):
    cid = lax.axis_index("c")
    sid = lax.axis_index("s")
    wid = sid * NC + cid
    eoff = wid * EPW

    sidxb = (sidxb0, sidxb1, sidxb2, sidxb3)
    didxb = (didxb0, didxb1, didxb2, didxb3)
    exbb = (exbb0, exbb1, exbb2, exbb3)
    rows = (rowsb0, rowsb1, rowsb2, rowsb3)
    sis = (si0, si1, si2, si3)
    dds = (dd0, dd1, dd2, dd3)
    ggs = (gg0, gg1, gg2, gg3)
    sss = (ss0, ss1, ss2, ss3)

    def _si_start(ch, b):
        pltpu.make_async_copy(src_hbm.at[pl.ds(eoff + ch * CH, CH)],
                              sidxb[b].at[0], sis[b]).start()
        pltpu.make_async_copy(exw_hbm.at[pl.ds(eoff + ch * CH, CH)],
                              exbb[b], sis[b]).start()

    def _si_wait(b):
        pltpu.make_async_copy(src_hbm.at[pl.ds(0, CH)],
                              sidxb[b].at[0], sis[b]).wait()
        pltpu.make_async_copy(exw_hbm.at[pl.ds(0, CH)],
                              exbb[b], sis[b]).wait()

    def _dd_start(ch, b):
        pltpu.make_async_copy(dst_hbm.at[pl.ds(eoff + ch * CH, CH)],
                              didxb[b].at[0], dds[b]).start()

    def _dd_wait(b):
        pltpu.make_async_copy(dst_hbm.at[pl.ds(0, CH)],
                              didxb[b].at[0], dds[b]).wait()

    def _g_start(b):
        pltpu.make_async_copy(h_hbm.at[sidxb[b].at[0]], rows[b],
                              ggs[b]).start()

    def _g_wait(b):
        pltpu.make_async_copy(h_hbm.at[sidxb[b].at[0]], rows[b],
                              ggs[b]).wait()

    def _s_start(b):
        pltpu.make_async_copy(rows[b], acc_s.at[didxb[b].at[0]],
                              sss[b]).start(add=True)

    def _s_drain(b):
        pltpu.make_async_copy(rows[b], acc_s.at[didxb[b].at[0]],
                              sss[b]).wait()

    zero16 = jnp.zeros((16,), jnp.float32)

    def _zr(k, _):
        for cc in range(8):
            rowsb0[k, pl.ds(cc * 16, 16)] = zero16
        return 0
    lax.fori_loop(0, CH, _zr, 0)
    base = sid * (NP // NS)
    for j in range(8):
        pltpu.sync_copy(rowsb0, acc_s.at[pl.ds(base + j * CH, CH)])

    for b in range(4):
        _si_start(b, b)
    _dd_start(0, 0)
    _dd_start(1, 1)
    _si_wait(0)
    _g_start(0)
    _si_wait(1)
    _g_start(1)
    plsc.subcore_barrier()

    def _slot(q, _):
        for r in range(4):
            ch = 4 * q + r
            b = r
            b2 = (r + 2) % 4
            b4 = r

            @pl.when((ch >= 2) & (ch - 2 < NCH))
            def _():
                _s_drain(b2)

            @pl.when(ch + 2 < NCH)
            def _():
                _dd_start(ch + 2, b2)
                _si_wait(b2)
                _g_start(b2)

            @pl.when(ch < NCH)
            def _():
                _g_wait(b)

                @plsc.parallel_loop(0, CH, step=1, unroll=8)
                def _scale(k):
                    xv = plsc.load_gather(exbb[b],
                                          [jnp.full((16,), k, jnp.int32)])
                    for cc in range(8):
                        rows[b][k, pl.ds(cc * 16, 16)] = (
                            rows[b][k, pl.ds(cc * 16, 16)] * xv)

            @pl.when(ch + 4 < NCH)
            def _():
                _si_start(ch + 4, b4)

            @pl.when(ch < NCH)
            def _():
                _dd_wait(b)
                _s_start(b)
        return 0
    lax.fori_loop(0, (NCH + 2 + 3) // 4, _slot, 0)

    plsc.subcore_barrier()
    pltpu.sync_copy(acc_s.at[pl.ds(base, NP // NS)],
                    parts_hbm.at[cid].at[pl.ds(base, NP // NS)])


def _rows(h, exw, src, dst):
    mesh = plsc.VectorSubcoreMesh(core_axis_name="c", subcore_axis_name="s")
    fn = pl.kernel(
        _rows_body,
        out_type=jax.ShapeDtypeStruct((NC, NP, D), jnp.float32),
        mesh=mesh,
        scratch_types=(
            [pltpu.VMEM((1, CH), jnp.int32) for _ in range(4)]
            + [pltpu.VMEM((1, CH), jnp.int32) for _ in range(4)]
            + [pltpu.VMEM((CH,), jnp.float32) for _ in range(4)]
            + [pltpu.VMEM((CH, D), jnp.float32) for _ in range(4)]
            + [pltpu.VMEM_SHARED((NP, D), jnp.float32)]
            + [pltpu.SemaphoreType.DMA for _ in range(16)]
        ),
        compiler_params=pltpu.CompilerParams(needs_layout_passes=False),
    )
    return fn(h, exw, src, dst)


def _fused_body(parts_ref, dparts_ref, hlp_ref, bcp_ref,
                wc_ref, wl_ref, asv_ref, adv_ref, bl_ref,
                h_ref, hl_ref, vecs_ref, cvec_ref, ma_ref, mb_ref):
    i = pl.program_id(0)
    acc = parts_ref[0] + parts_ref[1]
    den = dparts_ref[0] + dparts_ref[1]
    acc3 = acc.reshape(BLK // D, D, D)
    out3 = acc3 / (den[:, :, None] + 1e-16)
    hcur = out3.reshape(BLK, D) + bcp_ref[...][None, :] + hlp_ref[...]
    hcur = jnp.maximum(hcur, 0.0)
    h = jnp.dot(hcur, wc_ref[...], preferred_element_type=jnp.float32)
    h_ref[...] = h
    hl_ref[...] = (jnp.dot(hcur, wl_ref[...],
                           preferred_element_type=jnp.float32)
                   + bl_ref[...][None, :])
    asrc = jnp.dot(h, asv_ref[...], preferred_element_type=jnp.float32)
    adst = jnp.dot(h, adv_ref[...], preferred_element_type=jnp.float32)
    vecs_ref[0, :] = asrc
    vecs_ref[1, :] = adst

    @pl.when(i == 0)
    def _():
        ma_ref[0] = -1e30
        mb_ref[0] = -1e30

    ma_ref[0] = jnp.maximum(ma_ref[0], jnp.max(asrc))
    mb_ref[0] = jnp.maximum(mb_ref[0], jnp.max(adst))

    @pl.when(i == GRID - 1)
    def _():
        c = ma_ref[0] + mb_ref[0]
        c = jnp.where(c > 0, c, 0.2 * c)
        cvec_ref[...] = jnp.full((8, 128), c, jnp.float32)


def _fused(parts, dparts, hlp, bcp, Wc, Wl, a_src, a_dst, bl):
    return pl.pallas_call(
        _fused_body,
        grid=(GRID,),
        in_specs=[
            pl.BlockSpec((NC, BLK, D), lambda i: (0, i, 0)),
            pl.BlockSpec((NC, BLK // D, D), lambda i: (0, i, 0)),
            pl.BlockSpec((BLK, D), lambda i: (i, 0)),
            pl.BlockSpec((D,), lambda i: (0,)),
            pl.BlockSpec((D, D), lambda i: (0, 0)),
            pl.BlockSpec((D, D), lambda i: (0, 0)),
            pl.BlockSpec((D,), lambda i: (0,)),
            pl.BlockSpec((D,), lambda i: (0,)),
            pl.BlockSpec((D,), lambda i: (0,)),
        ],
        out_specs=[
            pl.BlockSpec((BLK, D), lambda i: (i, 0)),
            pl.BlockSpec((BLK, D), lambda i: (i, 0)),
            pl.BlockSpec((8, BLK), lambda i: (0, i)),
            pl.BlockSpec((8, 128), lambda i: (0, 0)),
        ],
        out_shape=[
            jax.ShapeDtypeStruct((NP, D), jnp.float32),
            jax.ShapeDtypeStruct((NP, D), jnp.float32),
            jax.ShapeDtypeStruct((8, NP), jnp.float32),
            jax.ShapeDtypeStruct((8, 128), jnp.float32),
        ],
        scratch_shapes=[pltpu.SMEM((1,), jnp.float32),
                        pltpu.SMEM((1,), jnp.float32)],
    )(parts, dparts, hlp, bcp, Wc, Wl, a_src, a_dst, bl)


def _combine_body(parts_ref, dparts_ref, hl_ref, bc_ref, out_ref, *, relu):
    acc = parts_ref[0] + parts_ref[1]
    den = dparts_ref[0] + dparts_ref[1]
    acc3 = acc.reshape(BLK // D, D, D)
    out3 = acc3 / (den[:, :, None] + 1e-16)
    out = out3.reshape(BLK, D) + bc_ref[...][None, :] + hl_ref[...]
    if relu:
        out = jnp.maximum(out, 0.0)
    out_ref[...] = out


def _combine(parts, dparts, hl, bc, relu):
    return pl.pallas_call(
        functools.partial(_combine_body, relu=relu),
        grid=(GRID,),
        in_specs=[
            pl.BlockSpec((NC, BLK, D), lambda i: (0, i, 0)),
            pl.BlockSpec((NC, BLK // D, D), lambda i: (0, i, 0)),
            pl.BlockSpec((BLK, D), lambda i: (i, 0)),
            pl.BlockSpec((D,), lambda i: (0,)),
        ],
        out_specs=pl.BlockSpec((BLK, D), lambda i: (i, 0)),
        out_shape=jax.ShapeDtypeStruct((NP, D), jnp.float32),
    )(parts, dparts, hl, bc)


def kernel(x, edge_index, Wc0, a_src0, a_dst0, bc0, Wl0, bl0,
           Wc1, a_src1, a_dst1, bc1, Wl1, bl1,
           Wc2, a_src2, a_dst2, bc2, Wl2, bl2):
    src = edge_index[0]
    dst = edge_index[1]
    xp = jnp.pad(x, ((0, NP - N), (0, 0)))
    hp, hl, vecs, cvec = _dense(xp, Wc0, Wl0, a_src0, a_dst0, bl0)
    exw, dparts = _escore(vecs, cvec, src, dst)
    parts = _rows(hp, exw, src, dst)
    for Wc, a_src, a_dst, bcp, Wl, bl in (
            (Wc1, a_src1, a_dst1, bc0, Wl1, bl1),
            (Wc2, a_src2, a_dst2, bc1, Wl2, bl2)):
        hp, hl, vecs, cvec = _fused(parts, dparts, hl, bcp,
                                    Wc, Wl, a_src, a_dst, bl)
        exw, dparts = _escore(vecs, cvec, src, dst)
        parts = _rows(hp, exw, src, dst)
    out = _combine(parts, dparts, hl, bc2, relu=False)
    return out[:N]


# speedup vs baseline: 1.0114x; 1.0114x over previous
import functools
import jax
import jax.numpy as jnp
from jax import lax
from jax.experimental import pallas as pl
from jax.experimental.pallas import tpu as pltpu
from jax.experimental.pallas import tpu_sc as plsc

N = 10000
NP = 10240
D = 128
E = 320000
NC = 2
NS = 16
NW = NC * NS
EPW = E // NW
CH = 80
NCH = EPW // CH
NG = EPW // 16
BLK = 2048
GRID = NP // BLK


def _dense_body(x_ref, wc_ref, wl_ref, asv_ref, adv_ref, bl_ref,
                h_ref, hl_ref, vecs_ref, cvec_ref, ma_ref, mb_ref):
    i = pl.program_id(0)
    xb = x_ref[...]
    h = jnp.dot(xb, wc_ref[...], preferred_element_type=jnp.float32)
    h_ref[...] = h
    hl_ref[...] = (jnp.dot(xb, wl_ref[...], preferred_element_type=jnp.float32)
                   + bl_ref[...][None, :])
    asrc = jnp.dot(h, asv_ref[...], preferred_element_type=jnp.float32)
    adst = jnp.dot(h, adv_ref[...], preferred_element_type=jnp.float32)
    vecs_ref[0, :] = asrc
    vecs_ref[1, :] = adst

    @pl.when(i == 0)
    def _():
        ma_ref[0] = -1e30
        mb_ref[0] = -1e30

    ma_ref[0] = jnp.maximum(ma_ref[0], jnp.max(asrc))
    mb_ref[0] = jnp.maximum(mb_ref[0], jnp.max(adst))

    @pl.when(i == GRID - 1)
    def _():
        c = ma_ref[0] + mb_ref[0]
        c = jnp.where(c > 0, c, 0.2 * c)
        cvec_ref[...] = jnp.full((8, 128), c, jnp.float32)


def _dense(h_in, Wc, Wl, a_src, a_dst, bl):
    return pl.pallas_call(
        _dense_body,
        grid=(GRID,),
        in_specs=[
            pl.BlockSpec((BLK, D), lambda i: (i, 0)),
            pl.BlockSpec((D, D), lambda i: (0, 0)),
            pl.BlockSpec((D, D), lambda i: (0, 0)),
            pl.BlockSpec((D,), lambda i: (0,)),
            pl.BlockSpec((D,), lambda i: (0,)),
            pl.BlockSpec((D,), lambda i: (0,)),
        ],
        out_specs=[
            pl.BlockSpec((BLK, D), lambda i: (i, 0)),
            pl.BlockSpec((BLK, D), lambda i: (i, 0)),
            pl.BlockSpec((8, BLK), lambda i: (0, i)),
            pl.BlockSpec((8, 128), lambda i: (0, 0)),
        ],
        out_shape=[
            jax.ShapeDtypeStruct((NP, D), jnp.float32),
            jax.ShapeDtypeStruct((NP, D), jnp.float32),
            jax.ShapeDtypeStruct((8, NP), jnp.float32),
            jax.ShapeDtypeStruct((8, 128), jnp.float32),
        ],
        scratch_shapes=[pltpu.SMEM((1,), jnp.float32),
                        pltpu.SMEM((1,), jnp.float32)],
    )(h_in, Wc, Wl, a_src, a_dst, bl)


def _dyngather(x, idx):
    dnums = lax.GatherDimensionNumbers(
        offset_dims=(), collapsed_slice_dims=(0,), start_index_map=(0,))
    return lax.gather(x, idx[:, None], dnums, slice_sizes=(1,),
                      mode=lax.GatherScatterMode.PROMISE_IN_BOUNDS)


def _escore_body(vecs_hbm, cvec_hbm, src_hbm, dst_hbm,
                 exw_hbm, dparts_hbm,
                 tbl_a, tbl_b, sidx, didx2, exl, denom, iidx, gbuf, zbuf,
                 dsum_s, dsem):
    cid = lax.axis_index("c")
    sid = lax.axis_index("s")
    wid = sid * NC + cid
    eoff = wid * EPW

    pltpu.make_async_copy(vecs_hbm.at[0], tbl_a, dsem).start()
    pltpu.make_async_copy(vecs_hbm.at[1], tbl_b, dsem).start()
    pltpu.make_async_copy(cvec_hbm.at[0, pl.ds(0, 16)], gbuf, dsem).start()
    pltpu.make_async_copy(src_hbm.at[pl.ds(eoff, EPW)], sidx, dsem).start()

    def _ld_didx(ch, _):
        pltpu.make_async_copy(dst_hbm.at[pl.ds(eoff + ch * CH, CH)],
                              didx2.at[ch], dsem).start()
        return 0
    lax.fori_loop(0, NCH, _ld_didx, 0)

    zero16 = jnp.zeros((16,), jnp.float32)
    io16 = lax.iota(jnp.int32, 16)

    def _zr(k, _):
        for cc in range(8):
            denom[k, pl.ds(cc * 16, 16)] = zero16
        return 0
    lax.fori_loop(0, CH, _zr, 0)
    for k in range(8):
        for cc in range(8):
            zbuf[k, pl.ds(cc * 16, 16)] = zero16
    pltpu.sync_copy(zbuf, dsum_s.at[pl.ds(sid * 8, 8)])
    for j in range(5):
        iidx[0, pl.ds(j * 16, 16)] = io16 + j * 16

    pltpu.make_async_copy(vecs_hbm.at[0], tbl_a, dsem).wait()
    pltpu.make_async_copy(vecs_hbm.at[1], tbl_b, dsem).wait()
    pltpu.make_async_copy(cvec_hbm.at[0, pl.ds(0, 16)], gbuf, dsem).wait()
    pltpu.make_async_copy(src_hbm.at[pl.ds(0, EPW)], sidx, dsem).wait()

    def _dr_didx(ch, _):
        pltpu.make_async_copy(dst_hbm.at[pl.ds(0, CH)], didx2.at[0],
                              dsem).wait()
        return 0
    lax.fori_loop(0, NCH, _dr_didx, 0)
    cconst = gbuf[...]

    plsc.subcore_barrier()

    @plsc.parallel_loop(0, NG, step=1, unroll=4)
    def _grp(i):
        s16 = sidx[pl.ds(i * 16, 16)]
        row = i // 5
        col = (i % 5) * 16
        d16 = didx2[row, pl.ds(col, 16)]
        e = plsc.load_gather(tbl_a, [s16]) + plsc.load_gather(tbl_b, [d16])
        e = jnp.where(e > 0, e, 0.2 * e)
        ex = jnp.exp(e - cconst)
        exl[pl.ds(i * 16, 16)] = ex
        plsc.addupdate_scatter(denom, [d16 >> 7, d16 & 127], ex)

    pltpu.sync_copy(exl, exw_hbm.at[pl.ds(eoff, EPW)])
    pltpu.sync_copy(denom, dsum_s.at[iidx.at[0]], add=True)
    plsc.subcore_barrier()
    pltpu.sync_copy(dsum_s.at[pl.ds(sid * 8, 8)],
                    dparts_hbm.at[cid].at[pl.ds(sid * 8, 8)])


def _escore(vecs, cvec, src, dst):
    mesh = plsc.VectorSubcoreMesh(core_axis_name="c", subcore_axis_name="s")
    fn = pl.kernel(
        _escore_body,
        out_type=[
            jax.ShapeDtypeStruct((E,), jnp.float32),
            jax.ShapeDtypeStruct((NC, 128, D), jnp.float32),
        ],
        mesh=mesh,
        scratch_types=[
            pltpu.VMEM((NP,), jnp.float32),
            pltpu.VMEM((NP,), jnp.float32),
            pltpu.VMEM((EPW,), jnp.int32),
            pltpu.VMEM((NCH, CH), jnp.int32),
            pltpu.VMEM((EPW,), jnp.float32),
            pltpu.VMEM((CH, D), jnp.float32),
            pltpu.VMEM((1, CH), jnp.int32),
            pltpu.VMEM((16,), jnp.float32),
            pltpu.VMEM((8, D), jnp.float32),
            pltpu.VMEM_SHARED((128, D), jnp.float32),
            pltpu.SemaphoreType.DMA,
        ],
        compiler_params=pltpu.CompilerParams(needs_layout_passes=False),
    )
    return fn(vecs, cvec, src, dst)


def _rows_body(h_hbm, exw_hbm, src_hbm, dst_hbm,
               parts_hbm,
               sidxb0, sidxb1, sidxb2, sidxb3,
               didxb0, didxb1, didxb2, didxb3,
               exbb0, exbb1, exbb2, exbb3,
               rowsb0, rowsb1, rowsb2, rowsb3,
               acc_s,
               si0, si1, si2, si3, dd0, dd1, dd2, dd3,
               gg0, gg1, gg2, gg3, ss0, ss1, ss2, ss3):
    cid = lax.axis_index("c")
    sid = lax.axis_index("s")
    wid = sid * NC + cid
    eoff = wid * EPW

    sidxb = (sidxb0, sidxb1, sidxb2, sidxb3)
    didxb = (didxb0, didxb1, didxb2, didxb3)
    exbb = (exbb0, exbb1, exbb2, exbb3)
    rows = (rowsb0, rowsb1, rowsb2, rowsb3)
    sis = (si0, si1, si2, si3)
    dds = (dd0, dd1, dd2, dd3)
    ggs = (gg0, gg1, gg2, gg3)
    sss = (ss0, ss1, ss2, ss3)

    def _si_start(ch, b):
        pltpu.make_async_copy(src_hbm.at[pl.ds(eoff + ch * CH, CH)],
                              sidxb[b].at[0], sis[b]).start()
        pltpu.make_async_copy(exw_hbm.at[pl.ds(eoff + ch * CH, CH)],
                              exbb[b], sis[b]).start()

    def _si_wait(b):
        pltpu.make_async_copy(src_hbm.at[pl.ds(0, CH)],
                              sidxb[b].at[0], sis[b]).wait()
        pltpu.make_async_copy(exw_hbm.at[pl.ds(0, CH)],
                              exbb[b], sis[b]).wait()

    def _dd_start(ch, b):
        pltpu.make_async_copy(dst_hbm.at[pl.ds(eoff + ch * CH, CH)],
                              didxb[b].at[0], dds[b]).start()

    def _dd_wait(b):
        pltpu.make_async_copy(dst_hbm.at[pl.ds(0, CH)],
                              didxb[b].at[0], dds[b]).wait()

    def _g_start(b):
        pltpu.make_async_copy(h_hbm.at[sidxb[b].at[0]], rows[b],
                              ggs[b]).start()

    def _g_wait(b):
        pltpu.make_async_copy(h_hbm.at[sidxb[b].at[0]], rows[b],
                              ggs[b]).wait()

    def _s_start(b):
        pltpu.make_async_copy(rows[b], acc_s.at[didxb[b].at[0]],
                              sss[b]).start(add=True)

    def _s_drain(b):
        pltpu.make_async_copy(rows[b], acc_s.at[didxb[b].at[0]],
                              sss[b]).wait()

    zero16 = jnp.zeros((16,), jnp.float32)

    def _zr(k, _):
        for cc in range(8):
            rowsb0[k, pl.ds(cc * 16, 16)] = zero16
        return 0
    lax.fori_loop(0, CH, _zr, 0)
    base = sid * (NP // NS)
    for j in range(8):
        pltpu.sync_copy(rowsb0, acc_s.at[pl.ds(base + j * CH, CH)])

    for b in range(4):
        _si_start(b, b)
    _dd_start(0, 0)
    _dd_start(1, 1)
    _si_wait(0)
    _g_start(0)
    _si_wait(1)
    _g_start(1)
    plsc.subcore_barrier()

    def _slot(q, _):
        for r in range(4):
            ch = 4 * q + r
            b = r
            b2 = (r + 2) % 4
            b4 = r

            @pl.when((ch >= 2) & (ch - 2 < NCH))
            def _():
                _s_drain(b2)

            @pl.when(ch + 2 < NCH)
            def _():
                _dd_start(ch + 2, b2)
                _si_wait(b2)
                _g_start(b2)

            @pl.when(ch < NCH)
            def _():
                _g_wait(b)

                @plsc.parallel_loop(0, CH, step=1, unroll=4)
                def _scale(k):
                    xv = plsc.load_gather(exbb[b],
                                          [jnp.full((16,), k, jnp.int32)])
                    for cc in range(8):
                        rows[b][k, pl.ds(cc * 16, 16)] = (
                            rows[b][k, pl.ds(cc * 16, 16)] * xv)

            @pl.when(ch + 4 < NCH)
            def _():
                _si_start(ch + 4, b4)

            @pl.when(ch < NCH)
            def _():
                _dd_wait(b)
                _s_start(b)
        return 0
    lax.fori_loop(0, (NCH + 2 + 3) // 4, _slot, 0)

    plsc.subcore_barrier()
    pltpu.sync_copy(acc_s.at[pl.ds(base, NP // NS)],
                    parts_hbm.at[cid].at[pl.ds(base, NP // NS)])


def _rows(h, exw, src, dst):
    mesh = plsc.VectorSubcoreMesh(core_axis_name="c", subcore_axis_name="s")
    fn = pl.kernel(
        _rows_body,
        out_type=jax.ShapeDtypeStruct((NC, NP, D), jnp.float32),
        mesh=mesh,
        scratch_types=(
            [pltpu.VMEM((1, CH), jnp.int32) for _ in range(4)]
            + [pltpu.VMEM((1, CH), jnp.int32) for _ in range(4)]
            + [pltpu.VMEM((CH,), jnp.float32) for _ in range(4)]
            + [pltpu.VMEM((CH, D), jnp.float32) for _ in range(4)]
            + [pltpu.VMEM_SHARED((NP, D), jnp.float32)]
            + [pltpu.SemaphoreType.DMA for _ in range(16)]
        ),
        compiler_params=pltpu.CompilerParams(needs_layout_passes=False),
    )
    return fn(h, exw, src, dst)


def _fused_body(parts_ref, dparts_ref, hlp_ref, bcp_ref,
                wc_ref, wl_ref, asv_ref, adv_ref, bl_ref,
                h_ref, hl_ref, vecs_ref, cvec_ref, ma_ref, mb_ref):
    i = pl.program_id(0)
    acc = parts_ref[0] + parts_ref[1]
    den = dparts_ref[0] + dparts_ref[1]
    acc3 = acc.reshape(BLK // D, D, D)
    out3 = acc3 / (den[:, :, None] + 1e-16)
    hcur = out3.reshape(BLK, D) + bcp_ref[...][None, :] + hlp_ref[...]
    hcur = jnp.maximum(hcur, 0.0)
    h = jnp.dot(hcur, wc_ref[...], preferred_element_type=jnp.float32)
    h_ref[...] = h
    hl_ref[...] = (jnp.dot(hcur, wl_ref[...],
                           preferred_element_type=jnp.float32)
                   + bl_ref[...][None, :])
    asrc = jnp.dot(h, asv_ref[...], preferred_element_type=jnp.float32)
    adst = jnp.dot(h, adv_ref[...], preferred_element_type=jnp.float32)
    vecs_ref[0, :] = asrc
    vecs_ref[1, :] = adst

    @pl.when(i == 0)
    def _():
        ma_ref[0] = -1e30
        mb_ref[0] = -1e30

    ma_ref[0] = jnp.maximum(ma_ref[0], jnp.max(asrc))
    mb_ref[0] = jnp.maximum(mb_ref[0], jnp.max(adst))

    @pl.when(i == GRID - 1)
    def _():
        c = ma_ref[0] + mb_ref[0]
        c = jnp.where(c > 0, c, 0.2 * c)
        cvec_ref[...] = jnp.full((8, 128), c, jnp.float32)


def _fused(parts, dparts, hlp, bcp, Wc, Wl, a_src, a_dst, bl):
    return pl.pallas_call(
        _fused_body,
        grid=(GRID,),
        in_specs=[
            pl.BlockSpec((NC, BLK, D), lambda i: (0, i, 0)),
            pl.BlockSpec((NC, BLK // D, D), lambda i: (0, i, 0)),
            pl.BlockSpec((BLK, D), lambda i: (i, 0)),
            pl.BlockSpec((D,), lambda i: (0,)),
            pl.BlockSpec((D, D), lambda i: (0, 0)),
            pl.BlockSpec((D, D), lambda i: (0, 0)),
            pl.BlockSpec((D,), lambda i: (0,)),
            pl.BlockSpec((D,), lambda i: (0,)),
            pl.BlockSpec((D,), lambda i: (0,)),
        ],
        out_specs=[
            pl.BlockSpec((BLK, D), lambda i: (i, 0)),
            pl.BlockSpec((BLK, D), lambda i: (i, 0)),
            pl.BlockSpec((8, BLK), lambda i: (0, i)),
            pl.BlockSpec((8, 128), lambda i: (0, 0)),
        ],
        out_shape=[
            jax.ShapeDtypeStruct((NP, D), jnp.float32),
            jax.ShapeDtypeStruct((NP, D), jnp.float32),
            jax.ShapeDtypeStruct((8, NP), jnp.float32),
            jax.ShapeDtypeStruct((8, 128), jnp.float32),
        ],
        scratch_shapes=[pltpu.SMEM((1,), jnp.float32),
                        pltpu.SMEM((1,), jnp.float32)],
    )(parts, dparts, hlp, bcp, Wc, Wl, a_src, a_dst, bl)


def _combine_body(parts_ref, dparts_ref, hl_ref, bc_ref, out_ref, *, relu):
    acc = parts_ref[0] + parts_ref[1]
    den = dparts_ref[0] + dparts_ref[1]
    acc3 = acc.reshape(BLK // D, D, D)
    out3 = acc3 / (den[:, :, None] + 1e-16)
    out = out3.reshape(BLK, D) + bc_ref[...][None, :] + hl_ref[...]
    if relu:
        out = jnp.maximum(out, 0.0)
    out_ref[...] = out


def _combine(parts, dparts, hl, bc, relu):
    return pl.pallas_call(
        functools.partial(_combine_body, relu=relu),
        grid=(GRID,),
        in_specs=[
            pl.BlockSpec((NC, BLK, D), lambda i: (0, i, 0)),
            pl.BlockSpec((NC, BLK // D, D), lambda i: (0, i, 0)),
            pl.BlockSpec((BLK, D), lambda i: (i, 0)),
            pl.BlockSpec((D,), lambda i: (0,)),
        ],
        out_specs=pl.BlockSpec((BLK, D), lambda i: (i, 0)),
        out_shape=jax.ShapeDtypeStruct((NP, D), jnp.float32),
    )(parts, dparts, hl, bc)


def kernel(x, edge_index, Wc0, a_src0, a_dst0, bc0, Wl0, bl0,
           Wc1, a_src1, a_dst1, bc1, Wl1, bl1,
           Wc2, a_src2, a_dst2, bc2, Wl2, bl2):
    src = edge_index[0]
    dst = edge_index[1]
    xp = jnp.pad(x, ((0, NP - N), (0, 0)))
    hp, hl, vecs, cvec = _dense(xp, Wc0, Wl0, a_src0, a_dst0, bl0)
    exw, dparts = _escore(vecs, cvec, src, dst)
    parts = _rows(hp, exw, src, dst)
    for Wc, a_src, a_dst, bcp, Wl, bl in (
            (Wc1, a_src1, a_dst1, bc0, Wl1, bl1),
            (Wc2, a_src2, a_dst2, bc1, Wl2, bl2)):
        hp, hl, vecs, cvec = _fused(parts, dparts, hl, bcp,
                                    Wc, Wl, a_src, a_dst, bl)
        exw, dparts = _escore(vecs, cvec, src, dst)
        parts = _rows(hp, exw, src, dst)
    out = _combine(parts, dparts, hl, bc2, relu=False)
    return out[:N]
